# Initial kernel scaffold; baseline (speedup 1.0000x reference)
import jax
import jax.numpy as jnp
from jax.experimental import pallas as pl


def kernel(x, pe, idxs_0, idxs_1, idxs_2):
    raise NotImplementedError("write your pallas kernel here")


# speedup vs baseline: 1.1040x; 1.1040x over previous
import functools

import jax
import jax.numpy as jnp
from jax import lax
from jax.experimental import pallas as pl
from jax.experimental.pallas import tpu as pltpu, tpu_sc as plsc

B = 4
S = 2048
D = 1024
NC, NS = 2, 16
NW = NC * NS
ROWS_PER_W = (B * S) // NW
CHUNK = 16
NCHUNK = ROWS_PER_W // CHUNK
WORKERS_PER_B = NW // B


def _sc_body(x_hbm, pe_hbm, i0_hbm, i1_hbm, i2_hbm, out_hbm,
             i0_v, i1_v, i2_v, xb0, xb1, pb0, pb1,
             sem_pe0, sem_pe1, sem_x0, sem_x1, sem_o0, sem_o1):
    wid = lax.axis_index("s") * NC + lax.axis_index("c")
    b = wid // WORKERS_PER_B
    s_base = (wid % WORKERS_PER_B) * ROWS_PER_W
    row_base = wid * ROWS_PER_W

    pltpu.sync_copy(i0_hbm, i0_v)
    pltpu.sync_copy(i1_hbm, i1_v)
    pltpu.sync_copy(i2_hbm, i2_v)

    lane = lax.iota(jnp.int32, 16)
    b16 = lax.broadcast(b * 16, (16,))
    b8 = lax.broadcast(b * 8, (16,))

    x_bufs = (xb0, xb1)
    pe_bufs = (pb0, pb1)
    sem_pe = (sem_pe0, sem_pe1)
    sem_x = (sem_x0, sem_x1)
    sem_o = (sem_o0, sem_o1)

    def make_t(c):
        s = s_base + c * CHUNK + lane
        i = lax.shift_right_logical(s, 7)
        j = lax.shift_right_logical(s, 3) & 15
        k = s & 7
        return (plsc.load_gather(i0_v, [b16 + i])
                + plsc.load_gather(i1_v, [b16 + j])
                + plsc.load_gather(i2_v, [b8 + k]))

    handles = {}

    def start_chunk(c):
        slot = c & 1
        if c >= 2:
            handles[("o", c - 2)].wait()
        t = make_t(c)
        handles[("pe", c)] = pltpu.async_copy(
            pe_hbm.at[t], pe_bufs[slot], sem_pe[slot])
        handles[("x", c)] = pltpu.async_copy(
            x_hbm.at[pl.ds(row_base + c * CHUNK, CHUNK)], x_bufs[slot],
            sem_x[slot])

    start_chunk(0)
    for c in range(NCHUNK):
        slot = c & 1
        if c + 1 < NCHUNK:
            start_chunk(c + 1)
        handles[("x", c)].wait()
        handles[("pe", c)].wait()
        xb = x_bufs[slot]
        pb = pe_bufs[slot]

        def add_body(g, _, xb=xb, pb=pb):
            col = g * 16
            for r in range(CHUNK):
                xb[r, pl.ds(col, 16)] = xb[r, pl.ds(col, 16)] + pb[r, pl.ds(col, 16)]
            return _

        lax.fori_loop(0, D // 16, add_body, None)
        handles[("o", c)] = pltpu.async_copy(
            xb, out_hbm.at[pl.ds(row_base + c * CHUNK, CHUNK)], sem_o[slot])
    handles[("o", NCHUNK - 2)].wait()
    handles[("o", NCHUNK - 1)].wait()


@jax.jit
def _sc_call(x2d, pe, idxs_0, idxs_1, idxs_2):
    mesh = plsc.VectorSubcoreMesh(
        core_axis_name="c", subcore_axis_name="s",
        num_cores=NC, num_subcores=NS)
    fn = pl.kernel(
        _sc_body,
        out_type=jax.ShapeDtypeStruct((B * S, D), jnp.float32),
        mesh=mesh,
        compiler_params=pltpu.CompilerParams(needs_layout_passes=False),
        scratch_types=[
            pltpu.VMEM((B * 16,), jnp.int32),
            pltpu.VMEM((B * 16,), jnp.int32),
            pltpu.VMEM((B * 8,), jnp.int32),
            pltpu.VMEM((CHUNK, D), jnp.float32),
            pltpu.VMEM((CHUNK, D), jnp.float32),
            pltpu.VMEM((CHUNK, D), jnp.float32),
            pltpu.VMEM((CHUNK, D), jnp.float32),
            pltpu.SemaphoreType.DMA,
            pltpu.SemaphoreType.DMA,
            pltpu.SemaphoreType.DMA,
            pltpu.SemaphoreType.DMA,
            pltpu.SemaphoreType.DMA,
            pltpu.SemaphoreType.DMA,
        ],
    )
    return fn(x2d, pe, idxs_0, idxs_1, idxs_2)


def kernel(x, pe, idxs_0, idxs_1, idxs_2):
    out = _sc_call(x.reshape(B * S, D), pe, idxs_0.reshape(-1),
                   idxs_1.reshape(-1), idxs_2.reshape(-1))
    return out.reshape(B, S, D)
